# baseline (speedup 1.0000x reference)
import jax
import jax.numpy as jnp
from jax import lax
from jax.experimental import pallas as pl
from jax.experimental.pallas import tpu as pltpu
from jax.experimental.pallas import tpu_sc as plsc

N = 10000
E = 320000
D = 128

NC = 2
NS = 16
L = 16

C = 128
CPW = 80
HALF = CPW // 2
NW = NC * NS
EP = NW * CPW * C
ROWS_A = (N // NS) // 8 * 8
ROWS_LAST = N - (NS - 1) * ROWS_A


def _matmul_body(x_ref, wt_ref, b_ref, h_ref):
    h_ref[...] = (
        jnp.dot(x_ref[...], wt_ref[...], preferred_element_type=jnp.float32)
        + b_ref[...]
    )


def _dense_h(x, wt, b2d):
    grid = 10
    blk = N // grid
    return pl.pallas_call(
        _matmul_body,
        grid=(grid,),
        in_specs=[
            pl.BlockSpec((blk, D), lambda i: (i, 0)),
            pl.BlockSpec((D, D), lambda i: (0, 0)),
            pl.BlockSpec((1, D), lambda i: (0, 0)),
        ],
        out_specs=pl.BlockSpec((blk, D), lambda i: (i, 0)),
        out_shape=jax.ShapeDtypeStruct((N, D), jnp.float32),
    )(x, wt, b2d)


def _add_body(a_ref, b_ref, o_ref):
    o_ref[...] = a_ref[...] + b_ref[...]


def _combine(partials):
    grid = 10
    blk = N // grid
    return pl.pallas_call(
        _add_body,
        grid=(grid,),
        in_specs=[
            pl.BlockSpec((blk, D), lambda i: (i, 0)),
            pl.BlockSpec((blk, D), lambda i: (i + grid, 0)),
        ],
        out_specs=pl.BlockSpec((blk, D), lambda i: (i, 0)),
        out_shape=jax.ShapeDtypeStruct((N, D), jnp.float32),
    )(partials, partials)


def _sc_body(h_hbm, rows_hbm, cols_hbm, vals_hbm, out_hbm,
             cols_v, rows_v, vals_v, rows_buf, acc_sh,
             sg0, sg1, ss0, ss1, sp0, sp1, sp2):
    semg = [sg0, sg1]
    sems = [ss0, ss1]
    c = lax.axis_index("c")
    s = lax.axis_index("s")
    wid = s * NC + c
    start = wid * CPW

    pf0 = pltpu.async_copy(cols_hbm.at[pl.ds(start, HALF)], cols_v, sp0)
    pf1 = pltpu.async_copy(rows_hbm.at[pl.ds(start, HALF)], rows_v, sp1)
    pf2 = pltpu.async_copy(vals_hbm.at[pl.ds(start, HALF)], vals_v, sp2)

    zeros16 = jnp.zeros((L,), jnp.float32)
    zbuf = rows_buf.at[0]

    def _zero_row(r, _):
        for q in range(D // L):
            zbuf[r, pl.ds(q * L, L)] = zeros16
        return 0

    lax.fori_loop(0, C, _zero_row, 0)

    acc_base = s * ROWS_A

    for k in range(ROWS_A // C):
        pltpu.sync_copy(zbuf, acc_sh.at[pl.ds(acc_base + k * C, C)])
    tail0 = ROWS_A - (ROWS_A // C) * C
    pltpu.sync_copy(zbuf.at[pl.ds(0, tail0)],
                    acc_sh.at[pl.ds(acc_base + (ROWS_A // C) * C, tail0)])

    @pl.when(s == NS - 1)
    def _zero_extra():
        extra = ROWS_LAST - ROWS_A
        pltpu.sync_copy(zbuf.at[pl.ds(0, extra)],
                        acc_sh.at[pl.ds(acc_base + ROWS_A, extra)])

    plsc.subcore_barrier()
    pf0.wait()
    pf1.wait()
    pf2.wait()

    def _wait_g(b):
        pltpu.make_async_copy(h_hbm.at[pl.ds(0, C)], rows_buf.at[b], semg[b]
                              ).wait()

    def _wait_s(b):
        pltpu.make_async_copy(h_hbm.at[pl.ds(0, C)], rows_buf.at[b], sems[b]
                              ).wait()

    def _scale(j, b):
        rb = rows_buf.at[b]

        def _group(g, _):
            v16 = vals_v[j, pl.ds(g * L, L)]
            for e in range(L):
                r = g * L + e
                bval = jnp.broadcast_to(v16[e], (L,))
                for q in range(D // L):
                    sl = pl.ds(q * L, L)
                    rb[r, sl] = rb[r, sl] * bval
            return 0

        lax.fori_loop(0, C // L, _group, 0)

    def _gather(j, b):
        pltpu.async_copy(h_hbm.at[cols_v.at[j]], rows_buf.at[b], semg[b])

    def _scatter(j, b):
        pltpu.async_copy(rows_buf.at[b], acc_sh.at[rows_v.at[j]],
                         sems[b], add=True)

    for half in range(2):
        if half == 1:
            base = start + HALF
            pltpu.async_copy(cols_hbm.at[pl.ds(base, HALF)], cols_v, sp0)
            pltpu.async_copy(rows_hbm.at[pl.ds(base, HALF)], rows_v, sp1)
            pltpu.async_copy(vals_hbm.at[pl.ds(base, HALF)], vals_v, sp2)
            pltpu.make_async_copy(
                cols_hbm.at[pl.ds(base, HALF)], cols_v, sp0).wait()
            pltpu.make_async_copy(
                rows_hbm.at[pl.ds(base, HALF)], rows_v, sp1).wait()
            pltpu.make_async_copy(
                vals_hbm.at[pl.ds(base, HALF)], vals_v, sp2).wait()

        _gather(0, 0)
        _gather(1, 1)

        _wait_g(0)
        _scale(0, 0)
        _scatter(0, 0)

        def _pair(j2, _):
            j = 2 * j2 + 1
            _wait_g(1)
            _wait_s(0)
            _gather(j + 1, 0)
            _scale(j, 1)
            _scatter(j, 1)

            _wait_g(0)
            _wait_s(1)
            _gather(j + 2, 1)
            _scale(j + 1, 0)
            _scatter(j + 1, 0)
            return 0

        lax.fori_loop(0, (HALF - 2) // 2, _pair, 0)

        _wait_g(1)
        _wait_s(0)
        _scale(HALF - 1, 1)
        _scatter(HALF - 1, 1)
        _wait_s(1)

    plsc.subcore_barrier()

    out_base = c * N + acc_base
    for k in range(ROWS_A // C):
        pltpu.sync_copy(acc_sh.at[pl.ds(acc_base + k * C, C)],
                        out_hbm.at[pl.ds(out_base + k * C, C)])
    pltpu.sync_copy(acc_sh.at[pl.ds(acc_base + (ROWS_A // C) * C, tail0)],
                    out_hbm.at[pl.ds(out_base + (ROWS_A // C) * C, tail0)])

    @pl.when(s == NS - 1)
    def _write_extra():
        extra = ROWS_LAST - ROWS_A
        pltpu.sync_copy(acc_sh.at[pl.ds(acc_base + ROWS_A, extra)],
                        out_hbm.at[pl.ds(out_base + ROWS_A, extra)])


def _sc_aggregate(h, rows2d, cols2d, vals2d):
    mesh = plsc.VectorSubcoreMesh(core_axis_name="c", subcore_axis_name="s")
    return pl.kernel(
        _sc_body,
        out_type=jax.ShapeDtypeStruct((NC * N, D), jnp.float32),
        mesh=mesh,
        scratch_types=[
            pltpu.VMEM((HALF, C), jnp.int32),
            pltpu.VMEM((HALF, C), jnp.int32),
            pltpu.VMEM((HALF, C), jnp.float32),
            pltpu.VMEM((2, C, D), jnp.float32),
            pltpu.VMEM_SHARED((N, D), jnp.float32),
        ] + [pltpu.SemaphoreType.DMA] * 7,
    )(h, rows2d, cols2d, vals2d)


def kernel(x, A_indices, A_values, W, b):
    pad = EP - E
    rows = jnp.concatenate([A_indices[0], jnp.zeros((pad,), A_indices.dtype)])
    cols = jnp.concatenate([A_indices[1], jnp.zeros((pad,), A_indices.dtype)])
    vals = jnp.concatenate([A_values, jnp.zeros((pad,), A_values.dtype)])
    rows2d = rows.reshape(EP // C, C)
    cols2d = cols.reshape(EP // C, C)
    vals2d = vals.reshape(EP // C, C)
    h = _dense_h(x, W.T, b.reshape(1, D))
    partials = _sc_aggregate(h, rows2d, cols2d, vals2d)
    return _combine(partials)


# speedup vs baseline: 1.7478x; 1.7478x over previous
import jax
import jax.numpy as jnp
from jax import lax
from jax.experimental import pallas as pl
from jax.experimental.pallas import tpu as pltpu
from jax.experimental.pallas import tpu_sc as plsc

N = 10000
E = 320000
D = 128

NC = 2
NS = 16
L = 16

C = 128
NCHUNK = E // C
NW = NC * NS
CH_BASE = NCHUNK // NW
CH_REM = NCHUNK % NW
ROWS_A = (N // NS) // 8 * 8
ROWS_LAST = N - (NS - 1) * ROWS_A


def _matmul_body(x_ref, wt_ref, b_ref, h_ref):
    h_ref[...] = (
        jnp.dot(x_ref[...], wt_ref[...], preferred_element_type=jnp.float32)
        + b_ref[...]
    )


def _dense_h(x, wt, b2d):
    grid = 10
    blk = N // grid
    return pl.pallas_call(
        _matmul_body,
        grid=(grid,),
        in_specs=[
            pl.BlockSpec((blk, D), lambda i: (i, 0)),
            pl.BlockSpec((D, D), lambda i: (0, 0)),
            pl.BlockSpec((1, D), lambda i: (0, 0)),
        ],
        out_specs=pl.BlockSpec((blk, D), lambda i: (i, 0)),
        out_shape=jax.ShapeDtypeStruct((N, D), jnp.float32),
    )(x, wt, b2d)


def _add_body(a_ref, b_ref, o_ref):
    o_ref[...] = a_ref[...] + b_ref[...]


def _combine(partials):
    grid = 10
    blk = N // grid
    return pl.pallas_call(
        _add_body,
        grid=(grid,),
        in_specs=[
            pl.BlockSpec((blk, D), lambda i: (i, 0)),
            pl.BlockSpec((blk, D), lambda i: (i + grid, 0)),
        ],
        out_specs=pl.BlockSpec((blk, D), lambda i: (i, 0)),
        out_shape=jax.ShapeDtypeStruct((N, D), jnp.float32),
    )(partials, partials)


def _sc_body(h_hbm, rows_hbm, cols_hbm, vals_hbm, out_hbm,
             cols_idx, rows_idx, vals_v, rows_buf, acc_sh, sem,
             si0, si1, si2):
    c = lax.axis_index("c")
    s = lax.axis_index("s")
    wid = s * NC + c

    zeros16 = jnp.zeros((L,), jnp.float32)

    def _zero_row(r, _):
        for q in range(D // L):
            rows_buf[r, pl.ds(q * L, L)] = zeros16
        return 0

    lax.fori_loop(0, C, _zero_row, 0)

    acc_base = s * ROWS_A

    for k in range(ROWS_A // C):
        pltpu.sync_copy(rows_buf, acc_sh.at[pl.ds(acc_base + k * C, C)])
    tail0 = ROWS_A - (ROWS_A // C) * C
    pltpu.sync_copy(rows_buf.at[pl.ds(0, tail0)],
                    acc_sh.at[pl.ds(acc_base + (ROWS_A // C) * C, tail0)])

    @pl.when(s == NS - 1)
    def _zero_extra():
        extra = ROWS_LAST - ROWS_A
        pltpu.sync_copy(rows_buf.at[pl.ds(0, extra)],
                        acc_sh.at[pl.ds(acc_base + ROWS_A, extra)])

    plsc.subcore_barrier()

    start = wid * CH_BASE + jnp.minimum(wid, CH_REM)
    count = CH_BASE + jnp.where(wid < CH_REM, 1, 0)

    def _chunk(j, _):
        base = (start + j) * C
        d0 = pltpu.async_copy(cols_hbm.at[pl.ds(base, C)], cols_idx, si0)
        d1 = pltpu.async_copy(rows_hbm.at[pl.ds(base, C)], rows_idx, si1)
        d2 = pltpu.async_copy(vals_hbm.at[pl.ds(base, C)], vals_v, si2)
        d0.wait()
        d1.wait()
        d2.wait()
        pltpu.async_copy(h_hbm.at[cols_idx], rows_buf, sem).wait()

        def _group(g, _):
            v16 = vals_v[pl.ds(g * L, L)]
            for e in range(L):
                r = g * L + e
                bval = jnp.broadcast_to(v16[e], (L,))
                for q in range(D // L):
                    sl = pl.ds(q * L, L)
                    rows_buf[r, sl] = rows_buf[r, sl] * bval
            return 0

        lax.fori_loop(0, C // L, _group, 0)

        pltpu.sync_copy(rows_buf, acc_sh.at[rows_idx], add=True)
        return 0

    lax.fori_loop(0, count, _chunk, 0)
    plsc.subcore_barrier()

    out_base = c * N + acc_base
    for k in range(ROWS_A // C):
        pltpu.sync_copy(acc_sh.at[pl.ds(acc_base + k * C, C)],
                        out_hbm.at[pl.ds(out_base + k * C, C)])
    pltpu.sync_copy(acc_sh.at[pl.ds(acc_base + (ROWS_A // C) * C, tail0)],
                    out_hbm.at[pl.ds(out_base + (ROWS_A // C) * C, tail0)])

    @pl.when(s == NS - 1)
    def _write_extra():
        extra = ROWS_LAST - ROWS_A
        pltpu.sync_copy(acc_sh.at[pl.ds(acc_base + ROWS_A, extra)],
                        out_hbm.at[pl.ds(out_base + ROWS_A, extra)])


def _sc_aggregate(h, rows, cols, vals):
    mesh = plsc.VectorSubcoreMesh(core_axis_name="c", subcore_axis_name="s")
    return pl.kernel(
        _sc_body,
        out_type=jax.ShapeDtypeStruct((NC * N, D), jnp.float32),
        mesh=mesh,
        scratch_types=[
            pltpu.VMEM((C,), jnp.int32),
            pltpu.VMEM((C,), jnp.int32),
            pltpu.VMEM((C,), jnp.float32),
            pltpu.VMEM((C, D), jnp.float32),
            pltpu.VMEM_SHARED((N, D), jnp.float32),
        ] + [pltpu.SemaphoreType.DMA] * 4,
    )(h, rows, cols, vals)


def kernel(x, A_indices, A_values, W, b):
    rows = A_indices[0]
    cols = A_indices[1]
    h = _dense_h(x, W.T, b.reshape(1, D))
    partials = _sc_aggregate(h, rows, cols, A_values)
    return _combine(partials)
